# baseline (speedup 1.0000x reference)
import jax
import jax.numpy as jnp
from jax import lax
from jax.experimental import pallas as pl
from jax.experimental.pallas import tpu as pltpu
from jax.experimental.pallas import tpu_sc as plsc

_WAY = 64
_NC = 2
_NS = 16
_NW = _NC * _NS
_CHUNK = 128
_NBUF = 6
_N_SC = 16384


def _sc_segment_body(ctx_hbm, lbl_hbm, sum_out,
                     rows_v, idx_v, zs_v, acc_s, ld_sems, st_sems):
    cid = lax.axis_index("c")
    sid = lax.axis_index("s")
    wid = sid * _NC + cid
    rows_per_w = _N_SC // _NW
    nchunks = rows_per_w // _CHUNK
    stripe = _WAY // _NS

    def start_load(k):
        base = wid * rows_per_w + k * _CHUNK
        b = k % _NBUF
        return (
            pltpu.async_copy(lbl_hbm.at[pl.ds(base, _CHUNK)], idx_v.at[b],
                             ld_sems.at[b]),
            pltpu.async_copy(ctx_hbm.at[pl.ds(base, _CHUNK), :], rows_v.at[b],
                             ld_sems.at[b]),
        )

    loads = {}
    for k in range(min(_NBUF, nchunks)):
        loads[k] = start_load(k)

    zrow = jnp.zeros((16,), jnp.float32)

    def zfill(i, _):
        for j in range(8):
            zs_v[i, pl.ds(j * 16, 16)] = zrow
        return 0

    lax.fori_loop(0, stripe, zfill, 0)

    pltpu.sync_copy(zs_v, acc_s.at[pl.ds(sid * stripe, stripe), :])
    plsc.subcore_barrier()

    scats = {}
    for k in range(nchunks):
        b = k % _NBUF
        if k >= 2 and (k - 2) in scats:
            scats.pop(k - 2).wait()
        nxt = k + _NBUF - 2
        if nxt < nchunks and nxt >= _NBUF:
            loads[nxt] = start_load(nxt)
        for d in loads.pop(k):
            d.wait()
        scats[k] = pltpu.async_copy(rows_v.at[b], acc_s.at[idx_v.at[b]],
                                    st_sems.at[b], add=True)
    for k in sorted(scats):
        scats.pop(k).wait()

    plsc.subcore_barrier()

    @pl.when(sid == 0)
    def _writeout():
        pltpu.sync_copy(acc_s, sum_out.at[cid])


def _tc_seg_body(ctx_ref, lbl_ref, sum_out_ref, acc):
    j = pl.program_id(0)
    nb = pl.num_programs(0)
    blk = ctx_ref.shape[0]
    lbl = lbl_ref[0, 0, :]
    one_hot = jnp.where(
        lbl[:, None] == lax.broadcasted_iota(jnp.int32, (blk, _WAY), 1),
        1.0, 0.0)
    psum = lax.dot_general(one_hot, ctx_ref[...], (((0,), (0,)), ((), ())),
                           preferred_element_type=jnp.float32,
                           precision=lax.Precision.HIGHEST)

    @pl.when(j == 0)
    def _init():
        acc[...] = psum

    @pl.when(j > 0)
    def _accum():
        acc[...] += psum

    @pl.when(j == nb - 1)
    def _out():
        sum_out_ref[...] = acc[...]


def _tc_cnt_body(lbl_ref, cnt_out_ref, acc):
    j = pl.program_id(0)
    nb = pl.num_programs(0)
    nr, _, cb = lbl_ref.shape
    pcnt = jnp.zeros((1, _WAY), jnp.float32)
    for r in range(nr):
        lbl = lbl_ref[r, 0, :]
        one_hot = jnp.where(
            lbl[:, None] == lax.broadcasted_iota(jnp.int32, (cb, _WAY), 1),
            1.0, 0.0)
        pcnt += jnp.sum(one_hot, axis=0, keepdims=True)

    @pl.when(j == 0)
    def _init():
        acc[...] = pcnt

    @pl.when(j > 0)
    def _accum():
        acc[...] += pcnt

    @pl.when(j == nb - 1)
    def _out():
        cnt_out_ref[...] = acc[...]


def _tc_dist_body(sums_ref, tsum_ref, cnt_ref, tgt_ref, out_ref):
    sums = sums_ref[0] + sums_ref[1] + tsum_ref[...]
    cnt = cnt_ref[0]
    protos = sums / cnt[:, None]
    t = tgt_ref[...]
    dot = lax.dot_general(protos, t, (((1,), (1,)), ((), ())),
                          preferred_element_type=jnp.float32)
    t2 = jnp.sum(t * t, axis=1)
    p2 = jnp.sum(protos * protos, axis=1)
    out_ref[...] = 2.0 * dot - t2[None, :] - p2[:, None]


def _sc_segment(context_features, labels):
    d = context_features.shape[1]
    mesh = plsc.VectorSubcoreMesh(core_axis_name="c", subcore_axis_name="s",
                                  num_cores=_NC, num_subcores=_NS)
    sc_fn = pl.kernel(
        _sc_segment_body,
        out_type=jax.ShapeDtypeStruct((_NC, _WAY, d), jnp.float32),
        mesh=mesh,
        scratch_types=[
            pltpu.VMEM((_NBUF, _CHUNK, d), jnp.float32),
            pltpu.VMEM((_NBUF, _CHUNK), jnp.int32),
            pltpu.VMEM((_WAY // _NS, d), jnp.float32),
            pltpu.VMEM_SHARED((_WAY, d), jnp.float32),
            pltpu.SemaphoreType.DMA((_NBUF,)),
            pltpu.SemaphoreType.DMA((_NBUF,)),
        ],
    )
    return sc_fn(context_features, labels)


@jax.jit
def kernel(context_features, context_labels, target_features):
    n, d = context_features.shape
    nt = target_features.shape[0]
    labels = context_labels.astype(jnp.int32)

    sums = _sc_segment(context_features, labels)

    cb = 1024
    n_tc = n - _N_SC
    lbl3 = labels.reshape(n // cb, 1, cb)
    off = _N_SC // cb
    tsum = pl.pallas_call(
        _tc_seg_body,
        grid=(n_tc // cb,),
        in_specs=[
            pl.BlockSpec((cb, d), lambda j: (j + off, 0)),
            pl.BlockSpec((1, 1, cb), lambda j: (j + off, 0, 0)),
        ],
        out_specs=pl.BlockSpec((_WAY, d), lambda j: (0, 0)),
        out_shape=jax.ShapeDtypeStruct((_WAY, d), jnp.float32),
        scratch_shapes=[pltpu.VMEM((_WAY, d), jnp.float32)],
    )(context_features, lbl3)

    lb = 4
    cnt = pl.pallas_call(
        _tc_cnt_body,
        grid=(n // (lb * cb),),
        in_specs=[pl.BlockSpec((lb, 1, cb), lambda j: (j, 0, 0))],
        out_specs=pl.BlockSpec((1, _WAY), lambda j: (0, 0)),
        out_shape=jax.ShapeDtypeStruct((1, _WAY), jnp.float32),
        scratch_shapes=[pltpu.VMEM((1, _WAY), jnp.float32)],
    )(lbl3)

    tb = 1024
    logits_t = pl.pallas_call(
        _tc_dist_body,
        grid=(nt // tb,),
        in_specs=[
            pl.BlockSpec((_NC, _WAY, d), lambda i: (0, 0, 0)),
            pl.BlockSpec((_WAY, d), lambda i: (0, 0)),
            pl.BlockSpec((1, _WAY), lambda i: (0, 0)),
            pl.BlockSpec((tb, d), lambda i: (i, 0)),
        ],
        out_specs=pl.BlockSpec((_WAY, tb), lambda i: (0, i)),
        out_shape=jax.ShapeDtypeStruct((_WAY, nt), jnp.float32),
    )(sums, tsum, cnt, target_features)
    return logits_t.T


# speedup vs baseline: 1.2143x; 1.2143x over previous
import jax
import jax.numpy as jnp
from jax import lax
from jax.experimental import pallas as pl
from jax.experimental.pallas import tpu as pltpu
from jax.experimental.pallas import tpu_sc as plsc

_WAY = 64
_NC = 2
_NS = 16
_NW = _NC * _NS
_CHUNK = 128
_NBUF = 6
_N_SC = 32768


def _sc_segment_body(ctx_hbm, lbl_hbm, sum_out,
                     rows_v, idx_v, zs_v, acc_s, ld_sems, st_sems):
    cid = lax.axis_index("c")
    sid = lax.axis_index("s")
    wid = sid * _NC + cid
    rows_per_w = _N_SC // _NW
    nchunks = rows_per_w // _CHUNK
    stripe = _WAY // _NS

    def start_load(k):
        base = wid * rows_per_w + k * _CHUNK
        b = k % _NBUF
        return (
            pltpu.async_copy(lbl_hbm.at[pl.ds(base, _CHUNK)], idx_v.at[b],
                             ld_sems.at[b]),
            pltpu.async_copy(ctx_hbm.at[pl.ds(base, _CHUNK), :], rows_v.at[b],
                             ld_sems.at[b]),
        )

    loads = {}
    for k in range(min(_NBUF, nchunks)):
        loads[k] = start_load(k)

    zrow = jnp.zeros((16,), jnp.float32)

    def zfill(i, _):
        for j in range(8):
            zs_v[i, pl.ds(j * 16, 16)] = zrow
        return 0

    lax.fori_loop(0, stripe, zfill, 0)

    pltpu.sync_copy(zs_v, acc_s.at[pl.ds(sid * stripe, stripe), :])
    plsc.subcore_barrier()

    scats = {}
    for k in range(nchunks):
        b = k % _NBUF
        if k >= 2 and (k - 2) in scats:
            scats.pop(k - 2).wait()
        nxt = k + _NBUF - 2
        if nxt < nchunks and nxt >= _NBUF:
            loads[nxt] = start_load(nxt)
        for d in loads.pop(k):
            d.wait()
        scats[k] = pltpu.async_copy(rows_v.at[b], acc_s.at[idx_v.at[b]],
                                    st_sems.at[b], add=True)
    for k in sorted(scats):
        scats.pop(k).wait()

    plsc.subcore_barrier()

    @pl.when(sid == 0)
    def _writeout():
        pltpu.sync_copy(acc_s, sum_out.at[cid])


def _tc_seg_body(ctx_ref, lbl_ref, sum_out_ref, acc):
    j = pl.program_id(0)
    nb = pl.num_programs(0)
    blk = ctx_ref.shape[0]
    lbl = lbl_ref[0, 0, :]
    one_hot = jnp.where(
        lbl[:, None] == lax.broadcasted_iota(jnp.int32, (blk, _WAY), 1),
        1.0, 0.0)
    psum = lax.dot_general(one_hot, ctx_ref[...], (((0,), (0,)), ((), ())),
                           preferred_element_type=jnp.float32,
                           precision=lax.Precision.HIGHEST)

    @pl.when(j == 0)
    def _init():
        acc[...] = psum

    @pl.when(j > 0)
    def _accum():
        acc[...] += psum

    @pl.when(j == nb - 1)
    def _out():
        sum_out_ref[...] = acc[...]


def _tc_cnt_body(lbl_ref, cnt_out_ref, acc):
    j = pl.program_id(0)
    nb = pl.num_programs(0)
    nr, _, cb = lbl_ref.shape
    pcnt = jnp.zeros((1, _WAY), jnp.float32)
    for r in range(nr):
        lbl = lbl_ref[r, 0, :]
        one_hot = jnp.where(
            lbl[:, None] == lax.broadcasted_iota(jnp.int32, (cb, _WAY), 1),
            1.0, 0.0)
        pcnt += jnp.sum(one_hot, axis=0, keepdims=True)

    @pl.when(j == 0)
    def _init():
        acc[...] = pcnt

    @pl.when(j > 0)
    def _accum():
        acc[...] += pcnt

    @pl.when(j == nb - 1)
    def _out():
        cnt_out_ref[...] = acc[...]


def _tc_dist_body(sums_ref, cnt_ref, tgt_ref, out_ref):
    sums = sums_ref[0] + sums_ref[1]
    cnt = cnt_ref[0]
    protos = sums / cnt[:, None]
    t = tgt_ref[...]
    dot = lax.dot_general(protos, t, (((1,), (1,)), ((), ())),
                          preferred_element_type=jnp.float32)
    t2 = jnp.sum(t * t, axis=1)
    p2 = jnp.sum(protos * protos, axis=1)
    out_ref[...] = 2.0 * dot - t2[None, :] - p2[:, None]


def _sc_segment(context_features, labels):
    d = context_features.shape[1]
    mesh = plsc.VectorSubcoreMesh(core_axis_name="c", subcore_axis_name="s",
                                  num_cores=_NC, num_subcores=_NS)
    sc_fn = pl.kernel(
        _sc_segment_body,
        out_type=jax.ShapeDtypeStruct((_NC, _WAY, d), jnp.float32),
        mesh=mesh,
        scratch_types=[
            pltpu.VMEM((_NBUF, _CHUNK, d), jnp.float32),
            pltpu.VMEM((_NBUF, _CHUNK), jnp.int32),
            pltpu.VMEM((_WAY // _NS, d), jnp.float32),
            pltpu.VMEM_SHARED((_WAY, d), jnp.float32),
            pltpu.SemaphoreType.DMA((_NBUF,)),
            pltpu.SemaphoreType.DMA((_NBUF,)),
        ],
    )
    return sc_fn(context_features, labels)


@jax.jit
def kernel(context_features, context_labels, target_features):
    n, d = context_features.shape
    nt = target_features.shape[0]
    labels = context_labels.astype(jnp.int32)

    sums = _sc_segment(context_features, labels)

    cb = 1024
    lbl3 = labels.reshape(n // cb, 1, cb)
    lb = 4
    cnt = pl.pallas_call(
        _tc_cnt_body,
        grid=(n // (lb * cb),),
        in_specs=[pl.BlockSpec((lb, 1, cb), lambda j: (j, 0, 0))],
        out_specs=pl.BlockSpec((1, _WAY), lambda j: (0, 0)),
        out_shape=jax.ShapeDtypeStruct((1, _WAY), jnp.float32),
        scratch_shapes=[pltpu.VMEM((1, _WAY), jnp.float32)],
    )(lbl3)

    tb = 1024
    logits_t = pl.pallas_call(
        _tc_dist_body,
        grid=(nt // tb,),
        in_specs=[
            pl.BlockSpec((_NC, _WAY, d), lambda i: (0, 0, 0)),
            pl.BlockSpec((1, _WAY), lambda i: (0, 0)),
            pl.BlockSpec((tb, d), lambda i: (i, 0)),
        ],
        out_specs=pl.BlockSpec((_WAY, tb), lambda i: (0, i)),
        out_shape=jax.ShapeDtypeStruct((_WAY, nt), jnp.float32),
    )(sums, cnt, target_features)
    return logits_t.T


# speedup vs baseline: 1.2187x; 1.0036x over previous
import jax
import jax.numpy as jnp
from jax import lax
from jax.experimental import pallas as pl
from jax.experimental.pallas import tpu as pltpu
from jax.experimental.pallas import tpu_sc as plsc

_WAY = 64
_NC = 2
_NS = 16
_NW = _NC * _NS
_CHUNK = 128
_NBUF = 6
_N_SC = 32768


def _sc_segment_body(ctx_hbm, lbl_hbm, sum_out,
                     rows_v, idx_v, zs_v, acc_s, ld_sems, st_sems):
    cid = lax.axis_index("c")
    sid = lax.axis_index("s")
    wid = sid * _NC + cid
    rows_per_w = _N_SC // _NW
    nchunks = rows_per_w // _CHUNK
    stripe = _WAY // _NS

    def start_load(k):
        base = wid * rows_per_w + k * _CHUNK
        b = k % _NBUF
        return (
            pltpu.async_copy(lbl_hbm.at[pl.ds(base, _CHUNK)], idx_v.at[b],
                             ld_sems.at[b]),
            pltpu.async_copy(ctx_hbm.at[pl.ds(base, _CHUNK), :], rows_v.at[b],
                             ld_sems.at[b]),
        )

    loads = {}
    for k in range(min(_NBUF, nchunks)):
        loads[k] = start_load(k)

    zrow = jnp.zeros((16,), jnp.float32)

    def zfill(i, _):
        for j in range(8):
            zs_v[i, pl.ds(j * 16, 16)] = zrow
        return 0

    lax.fori_loop(0, stripe, zfill, 0)

    pltpu.sync_copy(zs_v, acc_s.at[pl.ds(sid * stripe, stripe), :])
    plsc.subcore_barrier()

    scats = {}
    for k in range(nchunks):
        b = k % _NBUF
        if k >= 2 and (k - 2) in scats:
            scats.pop(k - 2).wait()
        nxt = k + _NBUF - 2
        if nxt < nchunks and nxt >= _NBUF:
            loads[nxt] = start_load(nxt)
        for d in loads.pop(k):
            d.wait()
        scats[k] = pltpu.async_copy(rows_v.at[b], acc_s.at[idx_v.at[b]],
                                    st_sems.at[b], add=True)
    for k in sorted(scats):
        scats.pop(k).wait()

    plsc.subcore_barrier()

    @pl.when(sid == 0)
    def _writeout():
        pltpu.sync_copy(acc_s, sum_out.at[cid])


def _tc_cnt_body(lbl_ref, cnt_out_ref, acc):
    j = pl.program_id(0)
    nb = pl.num_programs(0)
    nr, _, cb = lbl_ref.shape
    pcnt = jnp.zeros((1, _WAY), jnp.float32)
    for r in range(nr):
        lbl = lbl_ref[r, 0, :]
        one_hot = jnp.where(
            lbl[:, None] == lax.broadcasted_iota(jnp.int32, (cb, _WAY), 1),
            1.0, 0.0)
        pcnt += jnp.sum(one_hot, axis=0, keepdims=True)

    @pl.when(j == 0)
    def _init():
        acc[...] = pcnt

    @pl.when(j > 0)
    def _accum():
        acc[...] += pcnt

    @pl.when(j == nb - 1)
    def _out():
        cnt_out_ref[...] = acc[...]


def _tc_dist_body(sums_ref, cnt_ref, tgt_ref, out_ref):
    sums = sums_ref[0] + sums_ref[1]
    cnt = cnt_ref[0]
    protos = sums / cnt[:, None]
    t = tgt_ref[...]
    dot = lax.dot_general(protos, t, (((1,), (1,)), ((), ())),
                          preferred_element_type=jnp.float32)
    t2 = jnp.sum(t * t, axis=1)
    p2 = jnp.sum(protos * protos, axis=1)
    out_ref[...] = 2.0 * dot - t2[None, :] - p2[:, None]


def _sc_segment(context_features, labels):
    d = context_features.shape[1]
    mesh = plsc.VectorSubcoreMesh(core_axis_name="c", subcore_axis_name="s",
                                  num_cores=_NC, num_subcores=_NS)
    sc_fn = pl.kernel(
        _sc_segment_body,
        out_type=jax.ShapeDtypeStruct((_NC, _WAY, d), jnp.float32),
        mesh=mesh,
        scratch_types=[
            pltpu.VMEM((_NBUF, _CHUNK, d), jnp.float32),
            pltpu.VMEM((_NBUF, _CHUNK), jnp.int32),
            pltpu.VMEM((_WAY // _NS, d), jnp.float32),
            pltpu.VMEM_SHARED((_WAY, d), jnp.float32),
            pltpu.SemaphoreType.DMA((_NBUF,)),
            pltpu.SemaphoreType.DMA((_NBUF,)),
        ],
    )
    return sc_fn(context_features, labels)


@jax.jit
def kernel(context_features, context_labels, target_features):
    n, d = context_features.shape
    nt = target_features.shape[0]
    labels = context_labels.astype(jnp.int32)

    sums = _sc_segment(context_features, labels)

    cb = 1024
    lbl3 = labels.reshape(n // cb, 1, cb)
    lb = 4
    cnt = pl.pallas_call(
        _tc_cnt_body,
        grid=(n // (lb * cb),),
        in_specs=[pl.BlockSpec((lb, 1, cb), lambda j: (j, 0, 0))],
        out_specs=pl.BlockSpec((1, _WAY), lambda j: (0, 0)),
        out_shape=jax.ShapeDtypeStruct((1, _WAY), jnp.float32),
        scratch_shapes=[pltpu.VMEM((1, _WAY), jnp.float32)],
    )(lbl3)

    tb = 1024
    logits_t = pl.pallas_call(
        _tc_dist_body,
        grid=(nt // tb,),
        in_specs=[
            pl.BlockSpec((_NC, _WAY, d), lambda i: (0, 0, 0)),
            pl.BlockSpec((1, _WAY), lambda i: (0, 0)),
            pl.BlockSpec((tb, d), lambda i: (i, 0)),
        ],
        out_specs=pl.BlockSpec((_WAY, tb), lambda i: (0, i)),
        out_shape=jax.ShapeDtypeStruct((_WAY, nt), jnp.float32),
    )(sums, cnt, target_features)
    return logits_t.T
